# baseline (speedup 1.0000x reference)
import functools

import jax
import jax.numpy as jnp
from jax import lax
from jax.experimental import pallas as pl
from jax.experimental.pallas import tpu as pltpu
from jax.experimental.pallas import tpu_sc as plsc

NC = 2
NS = 16
NW = NC * NS

BATCH = 16384
D = 128
B_PER_W = BATCH // NW
CHUNK = 128
NCHUNK = B_PER_W // CHUNK
GROUPS = CHUNK // 16


def _tree_reduce_16(vecs, lane):
    def shuf(x, s):
        return x.at[jnp.bitwise_xor(lane, s)].get(mode="promise_in_bounds")

    for s in (1, 2, 4, 8):
        m = (lane & s) == 0
        vecs = [
            jnp.where(m, a, shuf(b, s)) + jnp.where(m, shuf(a, s), b)
            for a, b in zip(vecs[0::2], vecs[1::2])
        ]
    return vecs[0]


def _sc_body(uidx_hbm, iidx_hbm, u2e_hbm, i2e_hbm, out_hbm,
             uidx_v, iidx_v, u_rows, i_rows, out_v, sem_u, sem_i):
    wid = lax.axis_index("s") * NC + lax.axis_index("c")
    base = wid * B_PER_W

    pltpu.sync_copy(uidx_hbm.at[wid], uidx_v)
    pltpu.sync_copy(iidx_hbm.at[wid], iidx_v)

    lane = lax.iota(jnp.int32, 16)

    def issue(c):
        cu = pltpu.async_copy(u2e_hbm.at[uidx_v.at[c]], u_rows.at[c % 2],
                              sem_u)
        ci = pltpu.async_copy(i2e_hbm.at[iidx_v.at[c]], i_rows.at[c % 2],
                              sem_i)
        return cu, ci

    pending = issue(0)

    for c in range(NCHUNK):
        cu, ci = pending
        cu.wait()
        ci.wait()
        if c + 1 < NCHUNK:
            pending = issue(c + 1)
        ub = u_rows.at[c % 2]
        ib = i_rows.at[c % 2]

        def gbody(g, _, _ub=ub, _ib=ib, _c=c):
            accs = []
            for k in range(16):
                r = g * 16 + k
                acc = _ub[r, pl.ds(0, 16)] * _ib[r, pl.ds(0, 16)]
                for j in range(1, D // 16):
                    acc += (_ub[r, pl.ds(j * 16, 16)]
                            * _ib[r, pl.ds(j * 16, 16)])
                accs.append(acc)
            dots = _tree_reduce_16(accs, lane)
            out_v[pl.ds(_c * CHUNK + g * 16, 16)] = (
                1.0 / (1.0 + jnp.exp(-dots)))
            return 0

        lax.fori_loop(0, GROUPS, gbody, 0)

    pltpu.sync_copy(out_v, out_hbm.at[pl.ds(base, B_PER_W)])


@jax.jit
def _run(uidx, iidx, u2e, i2e):
    mesh = plsc.VectorSubcoreMesh(core_axis_name="c", subcore_axis_name="s")
    f = pl.kernel(
        _sc_body,
        mesh=mesh,
        out_type=jax.ShapeDtypeStruct((BATCH,), jnp.float32),
        scratch_types=[
            pltpu.VMEM((NCHUNK, CHUNK), jnp.int32),
            pltpu.VMEM((NCHUNK, CHUNK), jnp.int32),
            pltpu.VMEM((2, CHUNK, D), jnp.float32),
            pltpu.VMEM((2, CHUNK, D), jnp.float32),
            pltpu.VMEM((B_PER_W,), jnp.float32),
            pltpu.SemaphoreType.DMA,
            pltpu.SemaphoreType.DMA,
        ],
    )
    return f(uidx, iidx, u2e, i2e)


def kernel(user_inputs, u_item_inputs, u2e, i2e):
    uidx = user_inputs.reshape(NW, NCHUNK, CHUNK)
    iidx = u_item_inputs.reshape(NW, NCHUNK, CHUNK)
    return _run(uidx, iidx, u2e, i2e)
